# Initial kernel scaffold; baseline (speedup 1.0000x reference)
import jax
import jax.numpy as jnp
from jax.experimental import pallas as pl


def kernel(data, d, edge_index, A0, B0, C0, D0, b0, A1, B1, C1, D1, b1, A2, B2, C2, D2, b2, fW0, fb0, fW1, fb1, fW2, fb2, fW3, fb3):
    raise NotImplementedError("write your pallas kernel here")


# speedup vs baseline: 1.5556x; 1.5556x over previous
import functools

import jax
import jax.numpy as jnp
from jax import lax
from jax.experimental import pallas as pl
from jax.experimental.pallas import tpu as pltpu
from jax.experimental.pallas import tpu_sc as plsc

F32 = jnp.float32
I32 = jnp.int32

_DATA = 0
_DFLAT = 16
_L0 = 48
_L1 = 73
_L2 = 178
_FW0 = 288
_FB0 = 4384
_FW1 = 4448
_FB1 = 8544
_FW2 = 8608
_FB2 = 10656
_FW3 = 10688
_FB3 = 10752
_TOTAL = 10768

_mesh = plsc.VectorSubcoreMesh(core_axis_name="c", subcore_axis_name="s",
                               num_cores=2, num_subcores=16)


def _leaky(v):
    return jnp.where(v > 0, v, 0.01 * v)


@functools.partial(
    pl.kernel,
    mesh=_mesh,
    compiler_params=pltpu.CompilerParams(needs_layout_passes=False),
    out_type=jax.ShapeDtypeStruct((2,), F32),
    scratch_types=[
        pltpu.VMEM((_TOTAL,), F32),
        pltpu.VMEM((32,), I32),
        pltpu.VMEM((80,), F32),
        pltpu.VMEM((80,), F32),
        pltpu.VMEM((80,), F32),
        pltpu.VMEM((80,), F32),
        pltpu.VMEM((64,), F32),
        pltpu.VMEM((64,), F32),
        pltpu.VMEM((16,), F32),
        pltpu.SemaphoreType.DMA,
        pltpu.SemaphoreType.DMA,
    ],
)
def _sc_forward(flat_hbm, ei_hbm, out_hbm, flat_v, ei_v, bhT, chT, dhT,
                aggT, x_v, x2_v, out_v, sem0, sem1):
    wid = lax.axis_index("c") * 16 + lax.axis_index("s")

    @pl.when(wid == 0)
    def _():
        cp0 = pltpu.async_copy(flat_hbm, flat_v, sem0)
        cp1 = pltpu.async_copy(ei_hbm, ei_v, sem1)
        cp0.wait()
        cp1.wait()

        iota = lax.iota(I32, 16)
        iota0 = iota * 0
        src = ei_v[pl.ds(0, 16)]
        dst = ei_v[pl.ds(16, 16)]

        def bc(k):
            return plsc.load_gather(flat_v, [iota0 + k])

        def dense_cols(rows, w_off, d_in, d_out):
            cols = []
            for o in range(d_out):
                acc = rows[0] * bc(w_off + o)
                for i in range(1, d_in):
                    acc = acc + rows[i] * bc(w_off + i * d_out + o)
                cols.append(acc)
            return cols

        def gated_layer(rows, off, d_in):
            A = dense_cols(rows, off, d_in, 5)
            B = dense_cols(rows, off + d_in * 5, d_in, 5)
            C = dense_cols(rows, off + 2 * d_in * 5, d_in, 5)
            D = dense_cols(rows, off + 3 * d_in * 5, d_in, 5)
            b_off = off + 4 * d_in * 5
            zero = jnp.zeros((16,), F32)
            for f in range(5):
                bhT[pl.ds(16 * f, 16)] = B[f]
                chT[pl.ds(16 * f, 16)] = C[f]
                dhT[pl.ds(16 * f, 16)] = D[f]
                aggT[pl.ds(16 * f, 16)] = zero
            for f in range(5):
                sidx = src + 16 * f
                didx = dst + 16 * f
                bh = plsc.load_gather(bhT, [sidx])
                ch = plsc.load_gather(chT, [didx])
                dh = plsc.load_gather(dhT, [sidx])
                eta = 1.0 / (1.0 + jnp.exp(-(ch + dh)))
                plsc.addupdate_scatter(aggT, [didx], eta * bh)
            out_rows = []
            for f in range(5):
                v = A[f] + aggT[pl.ds(16 * f, 16)] + bc(b_off + f)
                out_rows.append(_leaky(v))
            return out_rows

        rows = [flat_v[pl.ds(_DATA, 16)]]
        rows = gated_layer(rows, _L0, 1)
        rows = gated_layer(rows, _L1, 5)
        rows = gated_layer(rows, _L2, 5)

        mask8 = iota < 8
        for f in range(5):
            plsc.store_scatter(x_v, [iota * 5 + f], rows[f], mask=mask8)
        plsc.store_scatter(x_v, [iota + 40], flat_v[pl.ds(_DFLAT, 16)])
        plsc.store_scatter(x_v, [iota + 56], flat_v[pl.ds(_DFLAT + 16, 16)],
                           mask=mask8)

        def mlp_layer(xref, w_off, b_off, d_in, d_out):
            nchunk = d_out // 16
            U = 4
            zero = jnp.zeros((16,), F32)
            init = []
            for u in range(U):
                for c in range(nchunk):
                    init.append(flat_v[pl.ds(b_off + 16 * c, 16)]
                                if u == 0 else zero)
            def body(ii, accs):
                accs = list(accs)
                for u in range(U):
                    i = ii * U + u
                    xv = plsc.load_gather(xref, [iota0 + i])
                    base = w_off + i * d_out
                    for c in range(nchunk):
                        w = plsc.load_gather(flat_v, [base + 16 * c + iota])
                        accs[u * nchunk + c] = accs[u * nchunk + c] + xv * w
                return tuple(accs)
            accs = lax.fori_loop(0, d_in // U, body, tuple(init))
            outs = []
            for c in range(nchunk):
                s = accs[c]
                for u in range(1, U):
                    s = s + accs[u * nchunk + c]
                outs.append(_leaky(s))
            return outs

        o1 = mlp_layer(x_v, _FW0, _FB0, 64, 64)
        for c in range(4):
            x2_v[pl.ds(16 * c, 16)] = o1[c]
        o2 = mlp_layer(x2_v, _FW1, _FB1, 64, 64)
        for c in range(4):
            x_v[pl.ds(16 * c, 16)] = o2[c]
        y = mlp_layer(x_v, _FW2, _FB2, 64, 32)

        sums = []
        for o in range(2):
            acc = jnp.zeros((16,), F32)
            for c in range(2):
                col = plsc.load_gather(
                    flat_v, [_FW3 + (iota + 16 * c) * 2 + o])
                acc = acc + y[c] * col
            sums.append(jnp.sum(acc))
        fb3v = plsc.load_gather(flat_v, [_FB3 + iota % 2])
        z = jnp.where(iota == 0, sums[0], sums[1]) + fb3v
        out_v[...] = 1.0 / (1.0 + jnp.exp(-z))
        pltpu.sync_copy(out_v.at[pl.ds(0, 2)], out_hbm)


def kernel(data, d, edge_index, A0, B0, C0, D0, b0, A1, B1, C1, D1, b1,
           A2, B2, C2, D2, b2, fW0, fb0, fW1, fb1, fW2, fb2, fW3, fb3):
    z8 = jnp.zeros((8,), F32)
    flat = jnp.concatenate([
        data.reshape(-1), z8,
        d.reshape(-1), z8,
        A0.reshape(-1), B0.reshape(-1), C0.reshape(-1), D0.reshape(-1), b0,
        A1.reshape(-1), B1.reshape(-1), C1.reshape(-1), D1.reshape(-1), b1,
        A2.reshape(-1), B2.reshape(-1), C2.reshape(-1), D2.reshape(-1), b2,
        jnp.zeros((5,), F32),
        fW0.reshape(-1), fb0,
        fW1.reshape(-1), fb1,
        fW2.reshape(-1), fb2,
        fW3.reshape(-1), fb3,
        jnp.zeros((14,), F32),
    ])
    ei = edge_index.reshape(-1)
    return _sc_forward(flat, ei)


# speedup vs baseline: 1.7811x; 1.1450x over previous
import functools

import jax
import jax.numpy as jnp
from jax import lax
from jax.experimental import pallas as pl
from jax.experimental.pallas import tpu as pltpu
from jax.experimental.pallas import tpu_sc as plsc

F32 = jnp.float32
I32 = jnp.int32

_mesh = plsc.VectorSubcoreMesh(core_axis_name="c", subcore_axis_name="s",
                               num_cores=2, num_subcores=16)


def _leaky(v):
    return jnp.where(v > 0, v, 0.01 * v)


_SCRATCH = dict(
    data_v=pltpu.VMEM((8, 1), F32),
    d_v=pltpu.VMEM((4, 6), F32),
    ei_v=pltpu.VMEM((2, 16), I32),
    A0_v=pltpu.VMEM((1, 5), F32), B0_v=pltpu.VMEM((1, 5), F32),
    C0_v=pltpu.VMEM((1, 5), F32), D0_v=pltpu.VMEM((1, 5), F32),
    b0_v=pltpu.VMEM((5,), F32),
    A1_v=pltpu.VMEM((5, 5), F32), B1_v=pltpu.VMEM((5, 5), F32),
    C1_v=pltpu.VMEM((5, 5), F32), D1_v=pltpu.VMEM((5, 5), F32),
    b1_v=pltpu.VMEM((5,), F32),
    A2_v=pltpu.VMEM((5, 5), F32), B2_v=pltpu.VMEM((5, 5), F32),
    C2_v=pltpu.VMEM((5, 5), F32), D2_v=pltpu.VMEM((5, 5), F32),
    b2_v=pltpu.VMEM((5,), F32),
    fW0_v=pltpu.VMEM((64, 64), F32), fb0_v=pltpu.VMEM((64,), F32),
    fW1_v=pltpu.VMEM((64, 64), F32), fb1_v=pltpu.VMEM((64,), F32),
    fW2_v=pltpu.VMEM((64, 32), F32), fb2_v=pltpu.VMEM((32,), F32),
    fW3_v=pltpu.VMEM((32, 2), F32), fb3_v=pltpu.VMEM((2,), F32),
    bhT=pltpu.VMEM((80,), F32),
    chT=pltpu.VMEM((80,), F32),
    dhT=pltpu.VMEM((80,), F32),
    aggT=pltpu.VMEM((80,), F32),
    x_v=pltpu.VMEM((64,), F32),
    x2_v=pltpu.VMEM((64,), F32),
    out_v=pltpu.VMEM((16,), F32),
    sem=pltpu.SemaphoreType.DMA,
)


@functools.partial(
    pl.kernel,
    mesh=_mesh,
    compiler_params=pltpu.CompilerParams(needs_layout_passes=False),
    out_type=jax.ShapeDtypeStruct((2,), F32),
    scratch_types=_SCRATCH,
)
def _sc_forward(data, d, edge_index, A0, B0, C0, D0, b0,
                A1, B1, C1, D1, b1, A2, B2, C2, D2, b2,
                fW0, fb0, fW1, fb1, fW2, fb2, fW3, fb3, out_hbm, *,
                data_v, d_v, ei_v, A0_v, B0_v, C0_v, D0_v, b0_v,
                A1_v, B1_v, C1_v, D1_v, b1_v, A2_v, B2_v, C2_v, D2_v, b2_v,
                fW0_v, fb0_v, fW1_v, fb1_v, fW2_v, fb2_v, fW3_v, fb3_v,
                bhT, chT, dhT, aggT, x_v, x2_v, out_v, sem):
    wid = lax.axis_index("c") * 16 + lax.axis_index("s")

    @pl.when(wid == 0)
    def _():
        pairs = [(data, data_v), (d, d_v), (edge_index, ei_v),
                 (A0, A0_v), (B0, B0_v), (C0, C0_v), (D0, D0_v), (b0, b0_v),
                 (A1, A1_v), (B1, B1_v), (C1, C1_v), (D1, D1_v), (b1, b1_v),
                 (A2, A2_v), (B2, B2_v), (C2, C2_v), (D2, D2_v), (b2, b2_v),
                 (fW0, fW0_v), (fb0, fb0_v), (fW1, fW1_v), (fb1, fb1_v),
                 (fW2, fW2_v), (fb2, fb2_v), (fW3, fW3_v), (fb3, fb3_v)]
        copies = [pltpu.async_copy(src, dst, sem) for src, dst in pairs]
        for cp in copies:
            cp.wait()

        iota = lax.iota(I32, 16)
        iota0 = iota * 0
        src = plsc.load_gather(ei_v, [iota0, iota])
        dst = plsc.load_gather(ei_v, [iota0 + 1, iota])
        mask8 = iota < 8

        def bc(ref, i, o):
            return plsc.load_gather(ref, [iota0 + i, iota0 + o])

        def dense_cols(rows, w_ref, d_in, d_out):
            cols = []
            for o in range(d_out):
                acc = rows[0] * bc(w_ref, 0, o)
                for i in range(1, d_in):
                    acc = acc + rows[i] * bc(w_ref, i, o)
                cols.append(acc)
            return cols

        def gated_layer(rows, Ar, Br, Cr, Dr, br, d_in):
            A = dense_cols(rows, Ar, d_in, 5)
            B = dense_cols(rows, Br, d_in, 5)
            C = dense_cols(rows, Cr, d_in, 5)
            D = dense_cols(rows, Dr, d_in, 5)
            zero = jnp.zeros((16,), F32)
            for f in range(5):
                bhT[pl.ds(16 * f, 16)] = B[f]
                chT[pl.ds(16 * f, 16)] = C[f]
                dhT[pl.ds(16 * f, 16)] = D[f]
                aggT[pl.ds(16 * f, 16)] = zero
            for f in range(5):
                sidx = src + 16 * f
                didx = dst + 16 * f
                bh = plsc.load_gather(bhT, [sidx])
                ch = plsc.load_gather(chT, [didx])
                dh = plsc.load_gather(dhT, [sidx])
                eta = 1.0 / (1.0 + jnp.exp(-(ch + dh)))
                plsc.addupdate_scatter(aggT, [didx], eta * bh)
            out_rows = []
            for f in range(5):
                bf = plsc.load_gather(br, [iota0 + f])
                v = A[f] + aggT[pl.ds(16 * f, 16)] + bf
                out_rows.append(_leaky(v))
            return out_rows

        h0 = plsc.load_gather(data_v, [iota % 8, iota0])
        h0 = jnp.where(mask8, h0, 0.0)
        rows = [h0]
        rows = gated_layer(rows, A0_v, B0_v, C0_v, D0_v, b0_v, 1)
        rows = gated_layer(rows, A1_v, B1_v, C1_v, D1_v, b1_v, 5)
        rows = gated_layer(rows, A2_v, B2_v, C2_v, D2_v, b2_v, 5)

        for f in range(5):
            plsc.store_scatter(x_v, [iota * 5 + f], rows[f], mask=mask8)
        dl0 = plsc.load_gather(d_v, [iota // 6, iota % 6])
        plsc.store_scatter(x_v, [iota + 40], dl0)
        dl1 = plsc.load_gather(d_v, [(iota + 16) // 6, (iota + 16) % 6],
                               mask=mask8)
        plsc.store_scatter(x_v, [iota + 56], dl1, mask=mask8)

        def mlp_layer(xref, w_ref, b_ref, d_in, d_out):
            nchunk = d_out // 16
            U = 4
            zero = jnp.zeros((16,), F32)
            init = []
            for u in range(U):
                for c in range(nchunk):
                    init.append(b_ref[pl.ds(16 * c, 16)]
                                if u == 0 else zero)
            def body(ii, accs):
                accs = list(accs)
                for u in range(U):
                    i = ii * U + u
                    xv = plsc.load_gather(xref, [iota0 + i])
                    for c in range(nchunk):
                        w = w_ref[i, pl.ds(16 * c, 16)]
                        accs[u * nchunk + c] = accs[u * nchunk + c] + xv * w
                return tuple(accs)
            accs = lax.fori_loop(0, d_in // U, body, tuple(init))
            outs = []
            for c in range(nchunk):
                s = accs[c]
                for u in range(1, U):
                    s = s + accs[u * nchunk + c]
                outs.append(_leaky(s))
            return outs

        o1 = mlp_layer(x_v, fW0_v, fb0_v, 64, 64)
        for c in range(4):
            x2_v[pl.ds(16 * c, 16)] = o1[c]
        o2 = mlp_layer(x2_v, fW1_v, fb1_v, 64, 64)
        for c in range(4):
            x_v[pl.ds(16 * c, 16)] = o2[c]
        y = mlp_layer(x_v, fW2_v, fb2_v, 64, 32)

        sums = []
        for o in range(2):
            acc = jnp.zeros((16,), F32)
            for cc in range(2):
                col = plsc.load_gather(fW3_v, [iota + 16 * cc, iota0 + o])
                acc = acc + y[cc] * col
            sums.append(jnp.sum(acc))
        fb3v = plsc.load_gather(fb3_v, [iota % 2])
        z = jnp.where(iota == 0, sums[0], sums[1]) + fb3v
        out_v[...] = 1.0 / (1.0 + jnp.exp(-z))
        pltpu.sync_copy(out_v.at[pl.ds(0, 2)], out_hbm)


def kernel(data, d, edge_index, A0, B0, C0, D0, b0, A1, B1, C1, D1, b1,
           A2, B2, C2, D2, b2, fW0, fb0, fW1, fb1, fW2, fb2, fW3, fb3):
    return _sc_forward(data, d, edge_index, A0, B0, C0, D0, b0,
                       A1, B1, C1, D1, b1, A2, B2, C2, D2, b2,
                       fW0, fb0, fW1, fb1, fW2, fb2, fW3, fb3)


# speedup vs baseline: 1.8791x; 1.0550x over previous
import functools

import jax
import jax.numpy as jnp
from jax import lax
from jax.experimental import pallas as pl
from jax.experimental.pallas import tpu as pltpu
from jax.experimental.pallas import tpu_sc as plsc

F32 = jnp.float32
I32 = jnp.int32

_mesh = plsc.VectorSubcoreMesh(core_axis_name="c", subcore_axis_name="s",
                               num_cores=1, num_subcores=16)


def _leaky(v):
    return jnp.where(v > 0, v, 0.01 * v)


_SCRATCH = dict(
    data_v=pltpu.VMEM((8, 1), F32),
    d_v=pltpu.VMEM((4, 6), F32),
    ei_v=pltpu.VMEM((2, 16), I32),
    A0_v=pltpu.VMEM((1, 5), F32), B0_v=pltpu.VMEM((1, 5), F32),
    C0_v=pltpu.VMEM((1, 5), F32), D0_v=pltpu.VMEM((1, 5), F32),
    b0_v=pltpu.VMEM((5,), F32),
    A1_v=pltpu.VMEM((5, 5), F32), B1_v=pltpu.VMEM((5, 5), F32),
    C1_v=pltpu.VMEM((5, 5), F32), D1_v=pltpu.VMEM((5, 5), F32),
    b1_v=pltpu.VMEM((5,), F32),
    A2_v=pltpu.VMEM((5, 5), F32), B2_v=pltpu.VMEM((5, 5), F32),
    C2_v=pltpu.VMEM((5, 5), F32), D2_v=pltpu.VMEM((5, 5), F32),
    b2_v=pltpu.VMEM((5,), F32),
    fW0_v=pltpu.VMEM((64, 64), F32), fb0_v=pltpu.VMEM((64,), F32),
    fW1_v=pltpu.VMEM((64, 64), F32), fb1_v=pltpu.VMEM((64,), F32),
    fW2_v=pltpu.VMEM((64, 32), F32), fb2_v=pltpu.VMEM((32,), F32),
    fW3_v=pltpu.VMEM((32, 2), F32), fb3_v=pltpu.VMEM((2,), F32),
    bhT=pltpu.VMEM((80,), F32),
    chT=pltpu.VMEM((80,), F32),
    dhT=pltpu.VMEM((80,), F32),
    aggT=pltpu.VMEM((80,), F32),
    x_v=pltpu.VMEM((64,), F32),
    x2_v=pltpu.VMEM((64,), F32),
    out_v=pltpu.VMEM((16,), F32),
    sem=pltpu.SemaphoreType.DMA,
)


@functools.partial(
    pl.kernel,
    mesh=_mesh,
    compiler_params=pltpu.CompilerParams(needs_layout_passes=False),
    out_type=jax.ShapeDtypeStruct((2,), F32),
    scratch_types=_SCRATCH,
)
def _sc_forward(data, d, edge_index, A0, B0, C0, D0, b0,
                A1, B1, C1, D1, b1, A2, B2, C2, D2, b2,
                fW0, fb0, fW1, fb1, fW2, fb2, fW3, fb3, out_hbm, *,
                data_v, d_v, ei_v, A0_v, B0_v, C0_v, D0_v, b0_v,
                A1_v, B1_v, C1_v, D1_v, b1_v, A2_v, B2_v, C2_v, D2_v, b2_v,
                fW0_v, fb0_v, fW1_v, fb1_v, fW2_v, fb2_v, fW3_v, fb3_v,
                bhT, chT, dhT, aggT, x_v, x2_v, out_v, sem):
    wid = lax.axis_index("s")

    @pl.when(wid == 0)
    def _():
        pairs = [(data, data_v), (d, d_v), (edge_index, ei_v),
                 (A0, A0_v), (B0, B0_v), (C0, C0_v), (D0, D0_v), (b0, b0_v),
                 (A1, A1_v), (B1, B1_v), (C1, C1_v), (D1, D1_v), (b1, b1_v),
                 (A2, A2_v), (B2, B2_v), (C2, C2_v), (D2, D2_v), (b2, b2_v),
                 (fW0, fW0_v), (fb0, fb0_v), (fW1, fW1_v), (fb1, fb1_v),
                 (fW2, fW2_v), (fb2, fb2_v), (fW3, fW3_v), (fb3, fb3_v)]
        copies = [pltpu.async_copy(src, dst, sem) for src, dst in pairs]
        for cp in copies:
            cp.wait()

        iota = lax.iota(I32, 16)
        iota0 = iota * 0
        src = plsc.load_gather(ei_v, [iota0, iota])
        dst = plsc.load_gather(ei_v, [iota0 + 1, iota])
        mask8 = iota < 8

        def bc(ref, i, o):
            return plsc.load_gather(ref, [iota0 + i, iota0 + o])

        def dense_cols(rows, w_ref, d_in, d_out):
            cols = []
            for o in range(d_out):
                acc = rows[0] * bc(w_ref, 0, o)
                for i in range(1, d_in):
                    acc = acc + rows[i] * bc(w_ref, i, o)
                cols.append(acc)
            return cols

        def gated_layer(rows, Ar, Br, Cr, Dr, br, d_in):
            A = dense_cols(rows, Ar, d_in, 5)
            B = dense_cols(rows, Br, d_in, 5)
            C = dense_cols(rows, Cr, d_in, 5)
            D = dense_cols(rows, Dr, d_in, 5)
            zero = jnp.zeros((16,), F32)
            for f in range(5):
                bhT[pl.ds(16 * f, 16)] = B[f]
                chT[pl.ds(16 * f, 16)] = C[f]
                dhT[pl.ds(16 * f, 16)] = D[f]
                aggT[pl.ds(16 * f, 16)] = zero
            for f in range(5):
                sidx = src + 16 * f
                didx = dst + 16 * f
                bh = plsc.load_gather(bhT, [sidx])
                ch = plsc.load_gather(chT, [didx])
                dh = plsc.load_gather(dhT, [sidx])
                eta = 1.0 / (1.0 + jnp.exp(-(ch + dh)))
                plsc.addupdate_scatter(aggT, [didx], eta * bh)
            out_rows = []
            for f in range(5):
                bf = plsc.load_gather(br, [iota0 + f])
                v = A[f] + aggT[pl.ds(16 * f, 16)] + bf
                out_rows.append(_leaky(v))
            return out_rows

        h0 = plsc.load_gather(data_v, [iota % 8, iota0])
        h0 = jnp.where(mask8, h0, 0.0)
        rows = [h0]
        rows = gated_layer(rows, A0_v, B0_v, C0_v, D0_v, b0_v, 1)
        rows = gated_layer(rows, A1_v, B1_v, C1_v, D1_v, b1_v, 5)
        rows = gated_layer(rows, A2_v, B2_v, C2_v, D2_v, b2_v, 5)

        for f in range(5):
            plsc.store_scatter(x_v, [iota * 5 + f], rows[f], mask=mask8)
        dl0 = plsc.load_gather(d_v, [iota // 6, iota % 6])
        plsc.store_scatter(x_v, [iota + 40], dl0)
        dl1 = plsc.load_gather(d_v, [(iota + 16) // 6, (iota + 16) % 6],
                               mask=mask8)
        plsc.store_scatter(x_v, [iota + 56], dl1, mask=mask8)

        def mlp_layer(xref, w_ref, b_ref, d_in, d_out):
            nchunk = d_out // 16
            U = 4
            zero = jnp.zeros((16,), F32)
            init = []
            for u in range(U):
                for c in range(nchunk):
                    init.append(b_ref[pl.ds(16 * c, 16)]
                                if u == 0 else zero)
            def body(ii, accs):
                accs = list(accs)
                for u in range(U):
                    i = ii * U + u
                    xv = plsc.load_gather(xref, [iota0 + i])
                    for c in range(nchunk):
                        w = w_ref[i, pl.ds(16 * c, 16)]
                        accs[u * nchunk + c] = accs[u * nchunk + c] + xv * w
                return tuple(accs)
            accs = lax.fori_loop(0, d_in // U, body, tuple(init))
            outs = []
            for c in range(nchunk):
                s = accs[c]
                for u in range(1, U):
                    s = s + accs[u * nchunk + c]
                outs.append(_leaky(s))
            return outs

        o1 = mlp_layer(x_v, fW0_v, fb0_v, 64, 64)
        for c in range(4):
            x2_v[pl.ds(16 * c, 16)] = o1[c]
        o2 = mlp_layer(x2_v, fW1_v, fb1_v, 64, 64)
        for c in range(4):
            x_v[pl.ds(16 * c, 16)] = o2[c]
        y = mlp_layer(x_v, fW2_v, fb2_v, 64, 32)

        sums = []
        for o in range(2):
            acc = jnp.zeros((16,), F32)
            for cc in range(2):
                col = plsc.load_gather(fW3_v, [iota + 16 * cc, iota0 + o])
                acc = acc + y[cc] * col
            sums.append(jnp.sum(acc))
        fb3v = plsc.load_gather(fb3_v, [iota % 2])
        z = jnp.where(iota == 0, sums[0], sums[1]) + fb3v
        out_v[...] = 1.0 / (1.0 + jnp.exp(-z))
        pltpu.sync_copy(out_v.at[pl.ds(0, 2)], out_hbm)


def kernel(data, d, edge_index, A0, B0, C0, D0, b0, A1, B1, C1, D1, b1,
           A2, B2, C2, D2, b2, fW0, fb0, fW1, fb1, fW2, fb2, fW3, fb3):
    return _sc_forward(data, d, edge_index, A0, B0, C0, D0, b0,
                       A1, B1, C1, D1, b1, A2, B2, C2, D2, b2,
                       fW0, fb0, fW1, fb1, fW2, fb2, fW3, fb3)
